# Initial kernel scaffold; baseline (speedup 1.0000x reference)
import jax
import jax.numpy as jnp
from jax.experimental import pallas as pl


def kernel(pred, mask, ind, target, pred_ab):
    raise NotImplementedError("write your pallas kernel here")


# speedup vs baseline: 1.5368x; 1.5368x over previous
import functools

import jax
import jax.numpy as jnp
from jax import lax
from jax.experimental import pallas as pl
from jax.experimental.pallas import tpu as pltpu
from jax.experimental.pallas import tpu_sc as plsc

B, C, H, W, K = 32, 2, 128, 128, 100
HW = H * W
KP = 128
NCHUNK = KP // 16


def _sc_body(pred_hbm, pab_hbm, ind_hbm, mask_hbm, tgt_hbm, out_hbm,
             idx_v, g0_v, g1_v, p0_v, p1_v, a0_v, a1_v, mask_v, tgt_v,
             out_v, sem):
    cid = lax.axis_index("c")
    sid = lax.axis_index("s")
    b = sid * 2 + cid

    pltpu.sync_copy(ind_hbm.at[b], idx_v)
    pltpu.sync_copy(mask_hbm.at[b], mask_v)
    pltpu.sync_copy(tgt_hbm.at[b], tgt_v)

    base = b * (C * HW)
    for i in range(NCHUNK):
        sl = pl.ds(i * 16, 16)
        idx = idx_v[sl]
        g0_v[sl] = idx + base
        g1_v[sl] = idx + (base + HW)

    cp0 = pltpu.async_copy(pred_hbm.at[g0_v], p0_v, sem)
    cp1 = pltpu.async_copy(pred_hbm.at[g1_v], p1_v, sem)
    cp2 = pltpu.async_copy(pab_hbm.at[g0_v], a0_v, sem)
    cp3 = pltpu.async_copy(pab_hbm.at[g1_v], a1_v, sem)
    cp0.wait()
    cp1.wait()
    cp2.wait()
    cp3.wait()

    acc = jnp.zeros((16,), jnp.float32)
    macc = jnp.zeros((16,), jnp.float32)
    for i in range(NCHUNK):
        sl = pl.ds(i * 16, 16)
        m = mask_v[sl]
        d0 = (p0_v[sl] - tgt_v[0, sl]) * m
        d1 = (p1_v[sl] - tgt_v[1, sl]) * m
        ad0 = jnp.abs(d0)
        ad1 = jnp.abs(d1)
        l0 = jnp.where(ad0 < 1.0, 0.5 * d0 * d0, ad0 - 0.5)
        l1 = jnp.where(ad1 < 1.0, 0.5 * d1 * d1, ad1 - 0.5)
        ab0 = jnp.maximum(a0_v[sl], 0.0) * m
        ab1 = jnp.maximum(a1_v[sl], 0.0) * m
        r = ab0 / (ab1 + 1e-8)
        wgt = jnp.where(r < 1.2, 1.0, 2.0)
        acc = acc + (l0 + l1) * wgt
        macc = macc + m

    out_v[0, :] = acc
    out_v[1, :] = macc
    pltpu.sync_copy(out_v, out_hbm.at[b])


@functools.lru_cache(maxsize=1)
def _build_sc_loss():
    return pl.kernel(
        _sc_body,
        out_type=jax.ShapeDtypeStruct((B, 2, 16), jnp.float32),
        mesh=plsc.VectorSubcoreMesh(core_axis_name="c", subcore_axis_name="s"),
        scratch_types=[
            pltpu.VMEM((KP,), jnp.int32),
            pltpu.VMEM((KP,), jnp.int32),
            pltpu.VMEM((KP,), jnp.int32),
            pltpu.VMEM((KP,), jnp.float32),
            pltpu.VMEM((KP,), jnp.float32),
            pltpu.VMEM((KP,), jnp.float32),
            pltpu.VMEM((KP,), jnp.float32),
            pltpu.VMEM((KP,), jnp.float32),
            pltpu.VMEM((C, KP), jnp.float32),
            pltpu.VMEM((2, 16), jnp.float32),
            pltpu.SemaphoreType.DMA,
        ],
    )


def kernel(pred, mask, ind, target, pred_ab):
    pred1d = pred.reshape(B * C * HW)
    pab1d = pred_ab.reshape(B * C * HW)
    ind_p = jnp.zeros((B, KP), jnp.int32).at[:, :K].set(ind.astype(jnp.int32))
    mask_p = jnp.zeros((B, KP), jnp.float32).at[:, :K].set(mask)
    tgt_p = jnp.zeros((B, C, KP), jnp.float32).at[:, :, :K].set(
        jnp.transpose(target, (0, 2, 1)))
    out = _build_sc_loss()(pred1d, pab1d, ind_p, mask_p, tgt_p)
    loss = jnp.sum(out[:, 0, :])
    msum = jnp.sum(out[:, 1, :])
    return loss / (msum + 1e-8)


# speedup vs baseline: 1.6182x; 1.0530x over previous
import functools

import jax
import jax.numpy as jnp
from jax import lax
from jax.experimental import pallas as pl
from jax.experimental.pallas import tpu as pltpu
from jax.experimental.pallas import tpu_sc as plsc

B, C, H, W, K = 32, 2, 128, 128, 100
HW = H * W
NCHUNK = 7
KW = 104
KB = 120
KG = NCHUNK * 16
TB = 224


def _sc_body(pred_hbm, pab_hbm, ind_hbm, mask_hbm, tgt_hbm, out_hbm,
             idx_v, g0_v, g1_v, p0_v, p1_v, a0_v, a1_v, mask_v, tgt_v,
             out_v, sem_i, sem_o, sem_g):
    cid = lax.axis_index("c")
    sid = lax.axis_index("s")
    b = sid * 2 + cid
    off = 4 * cid
    st = pl.multiple_of(b * K - off, 8)
    tst = pl.multiple_of(b * (K * C), 8)

    ci = pltpu.async_copy(ind_hbm.at[pl.ds(st, KW)], idx_v.at[pl.ds(0, KW)],
                          sem_i)
    cm = pltpu.async_copy(mask_hbm.at[pl.ds(st, KW)], mask_v.at[pl.ds(0, KW)],
                          sem_o)
    ct = pltpu.async_copy(tgt_hbm.at[pl.ds(tst, K * C)],
                          tgt_v.at[pl.ds(0, K * C)], sem_o)

    iota = lax.iota(jnp.int32, 16)
    perm0 = (iota * 2) % 16
    perm1 = perm0 + 1
    half = iota < 8
    ci.wait()

    base = b * (C * HW)
    for i in range(NCHUNK):
        sl = pl.ds(i * 16, 16)
        idx = idx_v[pl.ds(off + i * 16, 16)]
        idx = jnp.minimum(jnp.maximum(idx, 0), HW - 1)
        g0_v[sl] = idx + base
        g1_v[sl] = idx + (base + HW)

    cp0 = pltpu.async_copy(pred_hbm.at[g0_v], p0_v, sem_g)
    cp1 = pltpu.async_copy(pred_hbm.at[g1_v], p1_v, sem_g)
    cp2 = pltpu.async_copy(pab_hbm.at[g0_v], a0_v, sem_g)
    cp3 = pltpu.async_copy(pab_hbm.at[g1_v], a1_v, sem_g)
    cm.wait()
    ct.wait()
    cp0.wait()
    cp1.wait()
    cp2.wait()
    cp3.wait()

    acc = jnp.zeros((16,), jnp.float32)
    macc = jnp.zeros((16,), jnp.float32)
    for i in range(NCHUNK):
        sl = pl.ds(i * 16, 16)
        m = mask_v[pl.ds(off + i * 16, 16)]
        if (i + 1) * 16 > K:
            m = jnp.where(iota < K - i * 16, m, 0.0)
        ta = tgt_v[pl.ds(i * 32, 16)]
        tb = tgt_v[pl.ds(i * 32 + 16, 16)]
        t0 = jnp.where(half, ta[perm0], tb[perm0])
        t1 = jnp.where(half, ta[perm1], tb[perm1])
        d0 = (p0_v[sl] - t0) * m
        d1 = (p1_v[sl] - t1) * m
        ad0 = jnp.abs(d0)
        ad1 = jnp.abs(d1)
        l0 = jnp.where(ad0 < 1.0, 0.5 * d0 * d0, ad0 - 0.5)
        l1 = jnp.where(ad1 < 1.0, 0.5 * d1 * d1, ad1 - 0.5)
        ab0 = jnp.maximum(a0_v[sl], 0.0) * m
        ab1 = jnp.maximum(a1_v[sl], 0.0) * m
        r = ab0 / (ab1 + 1e-8)
        wgt = jnp.where(r < 1.2, 1.0, 2.0)
        acc = acc + (l0 + l1) * wgt
        macc = macc + m

    out_v[0, :] = acc
    out_v[1, :] = macc
    pltpu.sync_copy(out_v, out_hbm.at[b])


@functools.lru_cache(maxsize=1)
def _build_sc_loss():
    return pl.kernel(
        _sc_body,
        out_type=jax.ShapeDtypeStruct((B, 2, 16), jnp.float32),
        mesh=plsc.VectorSubcoreMesh(core_axis_name="c", subcore_axis_name="s"),
        scratch_types=[
            pltpu.VMEM((KB,), jnp.int32),
            pltpu.VMEM((KG,), jnp.int32),
            pltpu.VMEM((KG,), jnp.int32),
            pltpu.VMEM((KG,), jnp.float32),
            pltpu.VMEM((KG,), jnp.float32),
            pltpu.VMEM((KG,), jnp.float32),
            pltpu.VMEM((KG,), jnp.float32),
            pltpu.VMEM((KB,), jnp.float32),
            pltpu.VMEM((TB,), jnp.float32),
            pltpu.VMEM((2, 16), jnp.float32),
            pltpu.SemaphoreType.DMA,
            pltpu.SemaphoreType.DMA,
            pltpu.SemaphoreType.DMA,
        ],
    )


def kernel(pred, mask, ind, target, pred_ab):
    pred1d = pred.reshape(B * C * HW)
    pab1d = pred_ab.reshape(B * C * HW)
    ind1d = ind.astype(jnp.int32).reshape(B * K)
    mask1d = mask.reshape(B * K)
    tgt1d = target.reshape(B * K * C)
    out = _build_sc_loss()(pred1d, pab1d, ind1d, mask1d, tgt1d)
    loss = jnp.sum(out[:, 0, :])
    msum = jnp.sum(out[:, 1, :])
    return loss / (msum + 1e-8)


# speedup vs baseline: 1.7024x; 1.0521x over previous
import functools

import jax
import jax.numpy as jnp
from jax import lax
from jax.experimental import pallas as pl
from jax.experimental.pallas import tpu as pltpu
from jax.experimental.pallas import tpu_sc as plsc

B, C, H, W, K = 32, 2, 128, 128, 100
HW = H * W
NCHUNK = 7
KG = NCHUNK * 16
AW = 512


def _sc_body(pred_hbm, pab_hbm, aux_hbm, out_hbm,
             aux_v, g0_v, g1_v, p0_v, p1_v, a0_v, a1_v, out_v,
             sem_a, sem_g):
    cid = lax.axis_index("c")
    sid = lax.axis_index("s")
    b = sid * 2 + cid
    ast = pl.multiple_of(b * AW, 8)

    ca = pltpu.async_copy(aux_hbm.at[pl.ds(ast, AW)], aux_v, sem_a)

    iota = lax.iota(jnp.int32, 16)
    perm0 = (iota * 2) % 16
    perm1 = perm0 + 1
    half = iota < 8
    ca.wait()

    base = b * (C * HW)
    for i in range(NCHUNK):
        sl = pl.ds(i * 16, 16)
        idx = lax.bitcast_convert_type(aux_v[sl], jnp.int32)
        g0_v[sl] = idx + base
        g1_v[sl] = idx + (base + HW)

    cp0 = pltpu.async_copy(pred_hbm.at[g0_v], p0_v, sem_g)
    cp1 = pltpu.async_copy(pred_hbm.at[g1_v], p1_v, sem_g)
    cp2 = pltpu.async_copy(pab_hbm.at[g0_v], a0_v, sem_g)
    cp3 = pltpu.async_copy(pab_hbm.at[g1_v], a1_v, sem_g)
    cp0.wait()
    cp1.wait()
    cp2.wait()
    cp3.wait()

    acc = jnp.zeros((16,), jnp.float32)
    macc = jnp.zeros((16,), jnp.float32)
    for i in range(NCHUNK):
        sl = pl.ds(i * 16, 16)
        m = aux_v[pl.ds(128 + i * 16, 16)]
        ta = aux_v[pl.ds(256 + i * 32, 16)]
        tb = aux_v[pl.ds(256 + i * 32 + 16, 16)]
        t0 = jnp.where(half, ta[perm0], tb[perm0])
        t1 = jnp.where(half, ta[perm1], tb[perm1])
        d0 = (p0_v[sl] - t0) * m
        d1 = (p1_v[sl] - t1) * m
        ad0 = jnp.abs(d0)
        ad1 = jnp.abs(d1)
        l0 = jnp.where(ad0 < 1.0, 0.5 * d0 * d0, ad0 - 0.5)
        l1 = jnp.where(ad1 < 1.0, 0.5 * d1 * d1, ad1 - 0.5)
        ab0 = jnp.maximum(a0_v[sl], 0.0) * m
        ab1 = jnp.maximum(a1_v[sl], 0.0) * m
        r = ab0 / (ab1 + 1e-8)
        wgt = jnp.where(r < 1.2, 1.0, 2.0)
        acc = acc + (l0 + l1) * wgt
        macc = macc + m

    out_v[0, :] = acc
    out_v[1, :] = macc
    pltpu.sync_copy(out_v, out_hbm.at[b])


@functools.lru_cache(maxsize=1)
def _build_sc_loss():
    return pl.kernel(
        _sc_body,
        out_type=jax.ShapeDtypeStruct((B, 2, 16), jnp.float32),
        mesh=plsc.VectorSubcoreMesh(core_axis_name="c", subcore_axis_name="s"),
        scratch_types=[
            pltpu.VMEM((AW,), jnp.float32),
            pltpu.VMEM((KG,), jnp.int32),
            pltpu.VMEM((KG,), jnp.int32),
            pltpu.VMEM((KG,), jnp.float32),
            pltpu.VMEM((KG,), jnp.float32),
            pltpu.VMEM((KG,), jnp.float32),
            pltpu.VMEM((KG,), jnp.float32),
            pltpu.VMEM((2, 16), jnp.float32),
            pltpu.SemaphoreType.DMA,
            pltpu.SemaphoreType.DMA,
        ],
    )


def kernel(pred, mask, ind, target, pred_ab):
    pred1d = pred.reshape(B * C * HW)
    pab1d = pred_ab.reshape(B * C * HW)
    ind_f = lax.bitcast_convert_type(ind.astype(jnp.int32), jnp.float32)
    row = jnp.concatenate(
        [
            jnp.pad(ind_f, ((0, 0), (0, 128 - K))),
            jnp.pad(mask, ((0, 0), (0, 128 - K))),
            target.reshape(B, K * C),
            jnp.zeros((B, AW - 256 - K * C), jnp.float32),
        ],
        axis=1,
    )
    aux = row.reshape(B * AW)
    out = _build_sc_loss()(pred1d, pab1d, aux)
    loss = jnp.sum(out[:, 0, :])
    msum = jnp.sum(out[:, 1, :])
    return loss / (msum + 1e-8)


# speedup vs baseline: 1.7670x; 1.0379x over previous
import functools

import jax
import jax.numpy as jnp
from jax import lax
from jax.experimental import pallas as pl
from jax.experimental.pallas import tpu as pltpu
from jax.experimental.pallas import tpu_sc as plsc

B, C, H, W, K = 32, 2, 128, 128, 100
HW = H * W
NCHUNK = 7
KG = NCHUNK * 16
AW = 512


def _sc_body(pred_hbm, pab_hbm, aux_hbm, out_hbm,
             aux_v, g0_v, g1_v, p0_v, p1_v, a0_v, a1_v, out_v,
             sem_a, sem_g):
    cid = lax.axis_index("c")
    sid = lax.axis_index("s")
    b = sid * 2 + cid
    ast = pl.multiple_of(b * AW, 8)

    ca = pltpu.async_copy(aux_hbm.at[pl.ds(ast, AW)], aux_v, sem_a)

    iota = lax.iota(jnp.int32, 16)
    perm0 = (iota * 2) % 16
    perm1 = perm0 + 1
    half = iota < 8
    ca.wait()

    base = b * (C * HW)
    for i in range(NCHUNK):
        sl = pl.ds(i * 16, 16)
        idx = aux_v[sl].astype(jnp.int32)
        g0_v[sl] = idx + base
        g1_v[sl] = idx + (base + HW)

    cp0 = pltpu.async_copy(pred_hbm.at[g0_v], p0_v, sem_g)
    cp1 = pltpu.async_copy(pred_hbm.at[g1_v], p1_v, sem_g)
    cp2 = pltpu.async_copy(pab_hbm.at[g0_v], a0_v, sem_g)
    cp3 = pltpu.async_copy(pab_hbm.at[g1_v], a1_v, sem_g)
    cp0.wait()
    cp1.wait()
    cp2.wait()
    cp3.wait()

    acc = jnp.zeros((16,), jnp.float32)
    macc = jnp.zeros((16,), jnp.float32)
    for i in range(NCHUNK):
        sl = pl.ds(i * 16, 16)
        m = aux_v[pl.ds(128 + i * 16, 16)]
        ta = aux_v[pl.ds(256 + i * 32, 16)]
        tb = aux_v[pl.ds(256 + i * 32 + 16, 16)]
        t0 = jnp.where(half, ta[perm0], tb[perm0])
        t1 = jnp.where(half, ta[perm1], tb[perm1])
        d0 = (p0_v[sl] - t0) * m
        d1 = (p1_v[sl] - t1) * m
        ad0 = jnp.abs(d0)
        ad1 = jnp.abs(d1)
        l0 = jnp.where(ad0 < 1.0, 0.5 * d0 * d0, ad0 - 0.5)
        l1 = jnp.where(ad1 < 1.0, 0.5 * d1 * d1, ad1 - 0.5)
        ab0 = jnp.maximum(a0_v[sl], 0.0) * m
        ab1 = jnp.maximum(a1_v[sl], 0.0) * m
        r = ab0 / (ab1 + 1e-8)
        wgt = jnp.where(r < 1.2, 1.0, 2.0)
        acc = acc + (l0 + l1) * wgt
        macc = macc + m

    out_v[0, :] = acc
    out_v[1, :] = macc
    pltpu.sync_copy(out_v, out_hbm.at[b])


@functools.lru_cache(maxsize=1)
def _build_sc_loss():
    return pl.kernel(
        _sc_body,
        out_type=jax.ShapeDtypeStruct((B, 2, 16), jnp.float32),
        mesh=plsc.VectorSubcoreMesh(core_axis_name="c", subcore_axis_name="s"),
        scratch_types=[
            pltpu.VMEM((AW,), jnp.float32),
            pltpu.VMEM((KG,), jnp.int32),
            pltpu.VMEM((KG,), jnp.int32),
            pltpu.VMEM((KG,), jnp.float32),
            pltpu.VMEM((KG,), jnp.float32),
            pltpu.VMEM((KG,), jnp.float32),
            pltpu.VMEM((KG,), jnp.float32),
            pltpu.VMEM((2, 16), jnp.float32),
            pltpu.SemaphoreType.DMA,
            pltpu.SemaphoreType.DMA,
        ],
    )


def kernel(pred, mask, ind, target, pred_ab):
    pred1d = pred.reshape(B * C * HW)
    pab1d = pred_ab.reshape(B * C * HW)
    ind_f = ind.astype(jnp.float32)
    row = jnp.concatenate(
        [
            jnp.pad(ind_f, ((0, 0), (0, 128 - K))),
            jnp.pad(mask, ((0, 0), (0, 128 - K))),
            target.reshape(B, K * C),
            jnp.zeros((B, AW - 256 - K * C), jnp.float32),
        ],
        axis=1,
    )
    aux = row.reshape(B * AW)
    out = _build_sc_loss()(pred1d, pab1d, aux)
    loss = jnp.sum(out[:, 0, :])
    msum = jnp.sum(out[:, 1, :])
    return loss / (msum + 1e-8)


# speedup vs baseline: 1.7786x; 1.0066x over previous
import functools

import jax
import jax.numpy as jnp
from jax import lax
from jax.experimental import pallas as pl
from jax.experimental.pallas import tpu as pltpu
from jax.experimental.pallas import tpu_sc as plsc

B, C, H, W, K = 32, 2, 128, 128, 100
HW = H * W
NCHUNK = 7
KG = NCHUNK * 16
AW = 512


def _sc_body(pred_hbm, pab_hbm, aux_hbm, out_hbm,
             aux_v, g0_v, g1_v, p0_v, p1_v, a0_v, a1_v, out_v,
             sem_a, sem_o, sem_g):
    cid = lax.axis_index("c")
    sid = lax.axis_index("s")
    b = sid * 2 + cid

    ci = pltpu.async_copy(aux_hbm.at[b, pl.ds(0, 128)],
                          aux_v.at[pl.ds(0, 128)], sem_a)
    cr = pltpu.async_copy(aux_hbm.at[b, pl.ds(128, AW - 128)],
                          aux_v.at[pl.ds(128, AW - 128)], sem_o)
    ci.wait()

    base = b * (C * HW)
    for i in range(NCHUNK):
        sl = pl.ds(i * 16, 16)
        idx = aux_v[sl].astype(jnp.int32)
        g0_v[sl] = idx + base
        g1_v[sl] = idx + (base + HW)

    cp0 = pltpu.async_copy(pred_hbm.at[g0_v], p0_v, sem_g)
    cp1 = pltpu.async_copy(pred_hbm.at[g1_v], p1_v, sem_g)
    cp2 = pltpu.async_copy(pab_hbm.at[g0_v], a0_v, sem_g)
    cp3 = pltpu.async_copy(pab_hbm.at[g1_v], a1_v, sem_g)
    cr.wait()
    cp0.wait()
    cp1.wait()
    cp2.wait()
    cp3.wait()

    acc = jnp.zeros((16,), jnp.float32)
    macc = jnp.zeros((16,), jnp.float32)
    for i in range(NCHUNK):
        sl = pl.ds(i * 16, 16)
        m = aux_v[pl.ds(128 + i * 16, 16)]
        t0 = aux_v[pl.ds(256 + i * 16, 16)]
        t1 = aux_v[pl.ds(384 + i * 16, 16)]
        d0 = (p0_v[sl] - t0) * m
        d1 = (p1_v[sl] - t1) * m
        ad0 = jnp.abs(d0)
        ad1 = jnp.abs(d1)
        l0 = jnp.where(ad0 < 1.0, 0.5 * d0 * d0, ad0 - 0.5)
        l1 = jnp.where(ad1 < 1.0, 0.5 * d1 * d1, ad1 - 0.5)
        ab0 = jnp.maximum(a0_v[sl], 0.0) * m
        ab1 = jnp.maximum(a1_v[sl], 0.0) * m
        r = ab0 / (ab1 + 1e-8)
        wgt = jnp.where(r < 1.2, 1.0, 2.0)
        acc = acc + (l0 + l1) * wgt
        macc = macc + m

    out_v[0, :] = acc
    out_v[1, :] = macc
    pltpu.sync_copy(out_v, out_hbm.at[b])


@functools.lru_cache(maxsize=1)
def _build_sc_loss():
    return pl.kernel(
        _sc_body,
        out_type=jax.ShapeDtypeStruct((B, 2, 16), jnp.float32),
        mesh=plsc.VectorSubcoreMesh(core_axis_name="c", subcore_axis_name="s"),
        scratch_types=[
            pltpu.VMEM((AW,), jnp.float32),
            pltpu.VMEM((KG,), jnp.int32),
            pltpu.VMEM((KG,), jnp.int32),
            pltpu.VMEM((KG,), jnp.float32),
            pltpu.VMEM((KG,), jnp.float32),
            pltpu.VMEM((KG,), jnp.float32),
            pltpu.VMEM((KG,), jnp.float32),
            pltpu.VMEM((2, 16), jnp.float32),
            pltpu.SemaphoreType.DMA,
            pltpu.SemaphoreType.DMA,
            pltpu.SemaphoreType.DMA,
        ],
    )


def kernel(pred, mask, ind, target, pred_ab):
    pred1d = pred.reshape(B * C * HW)
    pab1d = pred_ab.reshape(B * C * HW)
    ind_f = ind.astype(jnp.float32)
    row = jnp.concatenate(
        [
            jnp.pad(ind_f, ((0, 0), (0, 128 - K))),
            jnp.pad(mask, ((0, 0), (0, 128 - K))),
            jnp.pad(target[:, :, 0], ((0, 0), (0, 128 - K))),
            jnp.pad(target[:, :, 1], ((0, 0), (0, 128 - K))),
        ],
        axis=1,
    )
    out = _build_sc_loss()(pred1d, pab1d, row)
    loss = jnp.sum(out[:, 0, :])
    msum = jnp.sum(out[:, 1, :])
    return loss / (msum + 1e-8)
